# baseline (speedup 1.0000x reference)
import functools

import jax
import jax.numpy as jnp
from jax import lax
from jax.experimental import pallas as pl
from jax.experimental.pallas import tpu as pltpu
from jax.experimental.pallas import tpu_sc as plsc

F32 = jnp.float32
H = 128
TW = 256
NC = 2
NS = 16
NW = NC * NS
KE = 128


def _tables_body(x_ref, pe_ref, ws_ref, wr_ref, br_ref, s_ref, r_ref):
    inp = jnp.concatenate([x_ref[...], pe_ref[...]], axis=1)
    s_ref[...] = jnp.dot(inp, ws_ref[...], preferred_element_type=F32)
    r_ref[...] = jnp.dot(inp, wr_ref[...],
                         preferred_element_type=F32) + br_ref[...]


def _make_tables(x, pe, ws, wr, br, bn):
    n = x.shape[0]
    grid = n // bn
    return pl.pallas_call(
        _tables_body,
        grid=(grid,),
        in_specs=[
            pl.BlockSpec((bn, H), lambda i: (i, 0)),
            pl.BlockSpec((bn, H), lambda i: (i, 0)),
            pl.BlockSpec((2 * H, 2 * H), lambda i: (0, 0)),
            pl.BlockSpec((2 * H, 2 * H), lambda i: (0, 0)),
            pl.BlockSpec((1, 2 * H), lambda i: (0, 0)),
        ],
        out_specs=[
            pl.BlockSpec((bn, TW), lambda i: (i, 0)),
            pl.BlockSpec((bn, TW), lambda i: (i, 0)),
        ],
        out_shape=[
            jax.ShapeDtypeStruct((n, TW), F32),
            jax.ShapeDtypeStruct((n, TW), F32),
        ],
    )(x, pe, ws, wr, br)


def _gather_combine(stab, rtab, send, rec, px, py, pz):
    e = send.shape[0]
    n = px.shape[0]
    nblk = e // KE
    iters = (nblk + NW - 1) // NW
    mesh = plsc.VectorSubcoreMesh(core_axis_name="c", subcore_axis_name="s",
                                  num_cores=NC, num_subcores=NS)

    @functools.partial(
        pl.kernel,
        out_type=[
            jax.ShapeDtypeStruct((e, H), F32),
            jax.ShapeDtypeStruct((e, H), F32),
            jax.ShapeDtypeStruct((e,), F32),
        ],
        mesh=mesh,
        compiler_params=pltpu.CompilerParams(needs_layout_passes=False),
        scratch_types=[
            pltpu.VMEM((KE,), jnp.int32),
            pltpu.VMEM((KE,), jnp.int32),
            pltpu.VMEM((KE, TW), F32),
            pltpu.VMEM((KE, TW), F32),
            pltpu.VMEM((KE,), F32),
            pltpu.VMEM((n,), F32),
            pltpu.VMEM((n,), F32),
            pltpu.VMEM((n,), F32),
            pltpu.SemaphoreType.DMA,
            pltpu.SemaphoreType.DMA,
        ],
    )
    def k(stab_hbm, rtab_hbm, send_hbm, rec_hbm, px_hbm, py_hbm, pz_hbm,
          outm_hbm, outp_hbm, dsq_hbm, sidx, ridx, bufa, bufb, dsqb,
          pxv, pyv, pzv, sem1, sem2):
        wid = lax.axis_index("s") * NC + lax.axis_index("c")
        pltpu.sync_copy(px_hbm, pxv)
        pltpu.sync_copy(py_hbm, pyv)
        pltpu.sync_copy(pz_hbm, pzv)

        def block(it, _):
            b = it * NW + wid

            @pl.when(b < nblk)
            def _():
                base = b * KE
                pltpu.sync_copy(send_hbm.at[pl.ds(base, KE)], sidx)
                pltpu.sync_copy(rec_hbm.at[pl.ds(base, KE)], ridx)
                cpa = pltpu.async_copy(stab_hbm.at[sidx], bufa, sem1)
                cpb = pltpu.async_copy(rtab_hbm.at[ridx], bufb, sem2)

                def dsq_chunk(j, _):
                    sl = pl.ds(j * 16, 16)
                    sv = sidx[sl]
                    rv = ridx[sl]
                    dx = plsc.load_gather(pxv, [sv]) - plsc.load_gather(pxv, [rv])
                    dy = plsc.load_gather(pyv, [sv]) - plsc.load_gather(pyv, [rv])
                    dz = plsc.load_gather(pzv, [sv]) - plsc.load_gather(pzv, [rv])
                    dsqb[sl] = dx * dx + dy * dy + dz * dz
                    return 0

                lax.fori_loop(0, KE // 16, dsq_chunk, 0)
                cpa.wait()
                cpb.wait()

                def add_row(j, _):
                    for v in range(TW // 16):
                        sl = pl.ds(v * 16, 16)
                        bufa[j, sl] = bufa[j, sl] + bufb[j, sl]
                    return 0

                lax.fori_loop(0, KE, add_row, 0)
                pltpu.sync_copy(bufa.at[:, pl.ds(0, H)],
                                outm_hbm.at[pl.ds(base, KE)])
                pltpu.sync_copy(bufa.at[:, pl.ds(H, H)],
                                outp_hbm.at[pl.ds(base, KE)])
                pltpu.sync_copy(dsqb, dsq_hbm.at[pl.ds(base, KE)])
            return 0

        lax.fori_loop(0, iters, block, 0)

    return k(stab, rtab, send, rec, px, py, pz)


def _edge_body(pm_ref, pp_ref, dsq_ref, wd_ref, wp_ref, m2w_ref, m2b_ref,
               p2w_ref, p2b_ref, msg_ref, msgp_ref):
    dist = jnp.sqrt(dsq_ref[...] + 1e-12)
    zm = pm_ref[...] + dist * wd_ref[...]
    zp = pp_ref[...] + dist * wp_ref[...]
    hm = zm * jax.nn.sigmoid(zm)
    t1 = jnp.dot(hm, m2w_ref[...], preferred_element_type=F32) + m2b_ref[...]
    msg_ref[...] = t1 * jax.nn.sigmoid(t1)
    hp = jnp.tanh(zp)
    t2 = jnp.dot(hp, p2w_ref[...], preferred_element_type=F32) + p2b_ref[...]
    msgp_ref[...] = jnp.tanh(t2)


def _edge_mlp(h1m, h1p, dsqcol, wd, wp, m2w, m2b, p2w, p2b, be):
    e = h1m.shape[0]
    grid = e // be
    return pl.pallas_call(
        _edge_body,
        grid=(grid,),
        in_specs=[
            pl.BlockSpec((be, H), lambda i: (i, 0)),
            pl.BlockSpec((be, H), lambda i: (i, 0)),
            pl.BlockSpec((be, 1), lambda i: (i, 0)),
            pl.BlockSpec((1, H), lambda i: (0, 0)),
            pl.BlockSpec((1, H), lambda i: (0, 0)),
            pl.BlockSpec((H, H), lambda i: (0, 0)),
            pl.BlockSpec((1, H), lambda i: (0, 0)),
            pl.BlockSpec((H, H), lambda i: (0, 0)),
            pl.BlockSpec((1, H), lambda i: (0, 0)),
        ],
        out_specs=[
            pl.BlockSpec((be, H), lambda i: (i, 0)),
            pl.BlockSpec((be, H), lambda i: (i, 0)),
        ],
        out_shape=[
            jax.ShapeDtypeStruct((e, H), F32),
            jax.ShapeDtypeStruct((e, H), F32),
        ],
    )(h1m, h1p, dsqcol, wd, wp, m2w, m2b, p2w, p2b)


def _scatter_add(msg, msgp, rec, n):
    e = rec.shape[0]
    nblk = e // KE
    iters = (nblk + NS - 1) // NS
    zrows = 200
    nrblk = n // zrows
    riters = (nrblk + NS - 1) // NS
    mesh = plsc.VectorSubcoreMesh(core_axis_name="c", subcore_axis_name="s",
                                  num_cores=NC, num_subcores=NS)

    @functools.partial(
        pl.kernel,
        out_type=[
            jax.ShapeDtypeStruct((n, H), F32),
            jax.ShapeDtypeStruct((n, H), F32),
        ],
        mesh=mesh,
        scratch_types=[
            pltpu.VMEM((KE,), jnp.int32),
            pltpu.VMEM((KE, H), F32),
            pltpu.VMEM((zrows, H), F32),
            pltpu.VMEM_SHARED((n, H), F32),
        ],
    )
    def k(msg_hbm, msgp_hbm, rec_hbm, agg_hbm, aggp_hbm,
          ridx, mbuf, zbuf, acc):
        cid = lax.axis_index("c")
        sid = lax.axis_index("s")

        def zrow(j, _):
            for v in range(H // 16):
                zbuf[j, pl.ds(v * 16, 16)] = jnp.zeros((16,), F32)
            return 0

        lax.fori_loop(0, zrows, zrow, 0)

        def zblock(it, _):
            b = it * NS + sid

            @pl.when(b < nrblk)
            def _():
                pltpu.sync_copy(zbuf, acc.at[pl.ds(b * zrows, zrows)])
            return 0

        lax.fori_loop(0, riters, zblock, 0)
        plsc.subcore_barrier()

        def run(src_hbm):
            def block(it, _):
                b = it * NS + sid

                @pl.when(b < nblk)
                def _():
                    base = b * KE
                    pltpu.sync_copy(rec_hbm.at[pl.ds(base, KE)], ridx)
                    pltpu.sync_copy(src_hbm.at[pl.ds(base, KE)], mbuf)
                    pltpu.sync_copy(mbuf, acc.at[ridx], add=True)
                return 0

            lax.fori_loop(0, iters, block, 0)

        @pl.when(cid == 0)
        def _():
            run(msg_hbm)

        @pl.when(cid == 1)
        def _():
            run(msgp_hbm)

        plsc.subcore_barrier()

        def wblock(it, _):
            b = it * NS + sid

            @pl.when(b < nrblk)
            def _():
                sl = pl.ds(b * zrows, zrows)

                @pl.when(cid == 0)
                def _():
                    pltpu.sync_copy(acc.at[sl], agg_hbm.at[sl])

                @pl.when(cid == 1)
                def _():
                    pltpu.sync_copy(acc.at[sl], aggp_hbm.at[sl])
            return 0

        lax.fori_loop(0, riters, wblock, 0)

    return k(msg, msgp, rec)


def _update_body(x_ref, pe_ref, ag_ref, agp_ref, u1w_ref, u1b_ref, u2w_ref,
                 u2b_ref, q1w_ref, q1b_ref, q2w_ref, q2b_ref, up_ref, upp_ref):
    cat1 = jnp.concatenate([x_ref[...], pe_ref[...], ag_ref[...]], axis=1)
    z1 = jnp.dot(cat1, u1w_ref[...], preferred_element_type=F32) + u1b_ref[...]
    h1 = z1 * jax.nn.sigmoid(z1)
    up_ref[...] = jnp.dot(h1, u2w_ref[...],
                          preferred_element_type=F32) + u2b_ref[...]
    cat2 = jnp.concatenate([pe_ref[...], agp_ref[...]], axis=1)
    z2 = jnp.dot(cat2, q1w_ref[...], preferred_element_type=F32) + q1b_ref[...]
    h2 = jnp.tanh(z2)
    upp_ref[...] = jnp.tanh(
        jnp.dot(h2, q2w_ref[...], preferred_element_type=F32) + q2b_ref[...])


def _node_update(x, pe, aggr, aggr_pos, u1w, u1b, u2w, u2b, q1w, q1b, q2w,
                 q2b, bn):
    n = x.shape[0]
    grid = n // bn
    row = lambda i: (i, 0)
    full = lambda shape: pl.BlockSpec(shape, lambda i: (0, 0))
    return pl.pallas_call(
        _update_body,
        grid=(grid,),
        in_specs=[
            pl.BlockSpec((bn, H), row),
            pl.BlockSpec((bn, H), row),
            pl.BlockSpec((bn, H), row),
            pl.BlockSpec((bn, H), row),
            full((3 * H, H)),
            full((1, H)),
            full((H, H)),
            full((1, H)),
            full((2 * H, H)),
            full((1, H)),
            full((H, H)),
            full((1, H)),
        ],
        out_specs=[
            pl.BlockSpec((bn, H), row),
            pl.BlockSpec((bn, H), row),
        ],
        out_shape=[
            jax.ShapeDtypeStruct((n, H), F32),
            jax.ShapeDtypeStruct((n, H), F32),
        ],
    )(x, pe, aggr, aggr_pos, u1w, u1b, u2w, u2b, q1w, q1b, q2w, q2b)


def kernel(x, pos, pe, edge_index, m1_W, m1_b, m2_W, m2_b, p1_W, p1_b, p2_W,
           p2_b, u1_W, u1_b, u2_W, u2_b, q1_W, q1_b, q2_W, q2_b):
    n = x.shape[0]
    e = edge_index.shape[1]
    send = edge_index[0].astype(jnp.int32)
    rec = edge_index[1].astype(jnp.int32)

    ws = jnp.zeros((2 * H, 2 * H), F32)
    ws = ws.at[:H, :H].set(m1_W[0:H])
    ws = ws.at[H:, :H].set(m1_W[H:2 * H])
    ws = ws.at[H:, H:].set(p1_W[0:H])
    wr = jnp.zeros((2 * H, 2 * H), F32)
    wr = wr.at[:H, :H].set(m1_W[2 * H:3 * H])
    wr = wr.at[H:, :H].set(m1_W[3 * H:4 * H])
    wr = wr.at[H:, H:].set(p1_W[H:2 * H])
    br = jnp.concatenate([m1_b, p1_b]).reshape(1, 2 * H)
    px = pos[:, 0].astype(F32)
    py = pos[:, 1].astype(F32)
    pz = pos[:, 2].astype(F32)

    stab, rtab = _make_tables(x, pe, ws, wr, br, bn=1000)
    h1m, h1p, dsq = _gather_combine(stab, rtab, send, rec, px, py, pz)
    dsqcol = dsq.reshape(e, 1)
    wd = m1_W[4 * H].reshape(1, H)
    wp = p1_W[2 * H].reshape(1, H)
    msg, msgp = _edge_mlp(h1m, h1p, dsqcol, wd, wp, m2_W, m2_b.reshape(1, H), p2_W,
                          p2_b.reshape(1, H), be=2560)
    aggr, aggr_pos = _scatter_add(msg, msgp, rec, n)
    update, update_pe = _node_update(
        x, pe, aggr, aggr_pos, u1_W, u1_b.reshape(1, H), u2_W,
        u2_b.reshape(1, H), q1_W, q1_b.reshape(1, H), q2_W,
        q2_b.reshape(1, H), bn=1000)
    return (update, update_pe)


# speedup vs baseline: 1.2401x; 1.2401x over previous
import functools

import jax
import jax.numpy as jnp
from jax import lax
from jax.experimental import pallas as pl
from jax.experimental.pallas import tpu as pltpu
from jax.experimental.pallas import tpu_sc as plsc

F32 = jnp.float32
H = 128
TW = 256
NC = 2
NS = 16
NW = NC * NS
KE = 128


def _tables_body(x_ref, pe_ref, ws_ref, wr_ref, br_ref, s_ref, r_ref):
    inp = jnp.concatenate([x_ref[...], pe_ref[...]], axis=1)
    s_ref[...] = jnp.dot(inp, ws_ref[...], preferred_element_type=F32)
    r_ref[...] = jnp.dot(inp, wr_ref[...],
                         preferred_element_type=F32) + br_ref[...]


def _make_tables(x, pe, ws, wr, br, bn):
    n = x.shape[0]
    grid = n // bn
    return pl.pallas_call(
        _tables_body,
        grid=(grid,),
        in_specs=[
            pl.BlockSpec((bn, H), lambda i: (i, 0)),
            pl.BlockSpec((bn, H), lambda i: (i, 0)),
            pl.BlockSpec((2 * H, 2 * H), lambda i: (0, 0)),
            pl.BlockSpec((2 * H, 2 * H), lambda i: (0, 0)),
            pl.BlockSpec((1, 2 * H), lambda i: (0, 0)),
        ],
        out_specs=[
            pl.BlockSpec((bn, TW), lambda i: (i, 0)),
            pl.BlockSpec((bn, TW), lambda i: (i, 0)),
        ],
        out_shape=[
            jax.ShapeDtypeStruct((n, TW), F32),
            jax.ShapeDtypeStruct((n, TW), F32),
        ],
    )(x, pe, ws, wr, br)


KG = 80


def _gather_combine(stab, rtab, send2d, rec2d, px, py, pz):
    nblk = send2d.shape[0]
    e = nblk * KG
    n = px.shape[0]
    bpt = nblk // NW
    mesh = plsc.VectorSubcoreMesh(core_axis_name="c", subcore_axis_name="s",
                                  num_cores=NC, num_subcores=NS)

    @functools.partial(
        pl.kernel,
        out_type=[
            jax.ShapeDtypeStruct((e, H), F32),
            jax.ShapeDtypeStruct((e, H), F32),
            jax.ShapeDtypeStruct((e,), F32),
        ],
        mesh=mesh,
        compiler_params=pltpu.CompilerParams(needs_layout_passes=False),
        scratch_types=[
            [pltpu.VMEM((KG,), jnp.int32) for _ in range(2)],
            [pltpu.VMEM((KG,), jnp.int32) for _ in range(2)],
            [pltpu.VMEM((KG, TW), F32) for _ in range(2)],
            [pltpu.VMEM((KG, TW), F32) for _ in range(2)],
            [pltpu.VMEM((KG,), F32) for _ in range(2)],
            pltpu.VMEM((n,), F32),
            pltpu.VMEM((n,), F32),
            pltpu.VMEM((n,), F32),
            [pltpu.SemaphoreType.DMA for _ in range(2)],
            [pltpu.SemaphoreType.DMA for _ in range(2)],
            [pltpu.SemaphoreType.DMA for _ in range(2)],
        ],
    )
    def k(stab_hbm, rtab_hbm, send_hbm, rec_hbm, px_hbm, py_hbm, pz_hbm,
          outm_hbm, outp_hbm, dsq_hbm, sidx, ridx, bufa, bufb, dsqb,
          pxv, pyv, pzv, sem_i, sem_g, sem_w):
        wid = lax.axis_index("s") * NC + lax.axis_index("c")
        b0 = wid * bpt
        pltpu.sync_copy(px_hbm, pxv)
        pltpu.sync_copy(py_hbm, pyv)
        pltpu.sync_copy(pz_hbm, pzv)

        def issue_idx(b, t):
            pltpu.async_copy(send_hbm.at[b], sidx[t], sem_i[t])
            pltpu.async_copy(rec_hbm.at[b], ridx[t], sem_i[t])

        def wait_idx(b, t):
            pltpu.make_async_copy(send_hbm.at[b], sidx[t], sem_i[t]).wait()
            pltpu.make_async_copy(rec_hbm.at[b], ridx[t], sem_i[t]).wait()

        def issue_gather(t):
            pltpu.async_copy(stab_hbm.at[sidx[t]], bufa[t], sem_g[t])
            pltpu.async_copy(rtab_hbm.at[ridx[t]], bufb[t], sem_g[t])

        def wait_gather(t):
            pltpu.make_async_copy(stab_hbm.at[sidx[t]], bufa[t],
                                  sem_g[t]).wait()
            pltpu.make_async_copy(rtab_hbm.at[ridx[t]], bufb[t],
                                  sem_g[t]).wait()

        def issue_writes(b, t):
            base = b * KG
            pltpu.async_copy(bufa[t].at[:, pl.ds(0, H)],
                             outm_hbm.at[pl.ds(base, KG)], sem_w[t])
            pltpu.async_copy(bufa[t].at[:, pl.ds(H, H)],
                             outp_hbm.at[pl.ds(base, KG)], sem_w[t])
            pltpu.async_copy(dsqb[t], dsq_hbm.at[pl.ds(base, KG)], sem_w[t])

        def wait_writes(b, t):
            base = b * KG
            pltpu.make_async_copy(bufa[t].at[:, pl.ds(0, H)],
                                  outm_hbm.at[pl.ds(base, KG)],
                                  sem_w[t]).wait()
            pltpu.make_async_copy(bufa[t].at[:, pl.ds(H, H)],
                                  outp_hbm.at[pl.ds(base, KG)],
                                  sem_w[t]).wait()
            pltpu.make_async_copy(dsqb[t], dsq_hbm.at[pl.ds(base, KG)],
                                  sem_w[t]).wait()

        def compute(b, t):
            def dsq_chunk(j, _):
                sl = pl.ds(j * 16, 16)
                sv = sidx[t][sl]
                rv = ridx[t][sl]
                dx = plsc.load_gather(pxv, [sv]) - plsc.load_gather(pxv, [rv])
                dy = plsc.load_gather(pyv, [sv]) - plsc.load_gather(pyv, [rv])
                dz = plsc.load_gather(pzv, [sv]) - plsc.load_gather(pzv, [rv])
                dsqb[t][sl] = dx * dx + dy * dy + dz * dz
                return 0

            lax.fori_loop(0, KG // 16, dsq_chunk, 0)

            def add_row(j, _):
                for v in range(TW // 16):
                    sl = pl.ds(v * 16, 16)
                    bufa[t][j, sl] = bufa[t][j, sl] + bufb[t][j, sl]
                return 0

            lax.fori_loop(0, KG, add_row, 0)

        pltpu.sync_copy(send_hbm.at[b0], sidx[0])
        pltpu.sync_copy(rec_hbm.at[b0], ridx[0])
        issue_gather(0)
        issue_idx(b0 + 1, 1)

        def step(it, _):
            b = b0 + it

            def body(s, t):
                @pl.when(it + 1 < bpt)
                def _():
                    wait_idx(b + 1, t)

                @pl.when(it >= 1)
                def _():
                    wait_writes(b - 1, t)

                @pl.when(it + 1 < bpt)
                def _():
                    issue_gather(t)
                wait_gather(s)
                compute(b, s)
                issue_writes(b, s)

                @pl.when(it + 2 < bpt)
                def _():
                    issue_idx(b + 2, s)

            @pl.when(it % 2 == 0)
            def _():
                body(0, 1)

            @pl.when(it % 2 == 1)
            def _():
                body(1, 0)
            return 0

        lax.fori_loop(0, bpt, step, 0)
        wait_writes(b0 + bpt - 1, (bpt - 1) % 2)

    return k(stab, rtab, send2d, rec2d, px, py, pz)


def _edge_body(pm_ref, pp_ref, dsq_ref, wd_ref, wp_ref, m2w_ref, m2b_ref,
               p2w_ref, p2b_ref, msg_ref, msgp_ref):
    dist = jnp.sqrt(dsq_ref[...] + 1e-12)
    zm = pm_ref[...] + dist * wd_ref[...]
    zp = pp_ref[...] + dist * wp_ref[...]
    hm = zm * jax.nn.sigmoid(zm)
    t1 = jnp.dot(hm, m2w_ref[...], preferred_element_type=F32) + m2b_ref[...]
    msg_ref[...] = t1 * jax.nn.sigmoid(t1)
    hp = jnp.tanh(zp)
    t2 = jnp.dot(hp, p2w_ref[...], preferred_element_type=F32) + p2b_ref[...]
    msgp_ref[...] = jnp.tanh(t2)


def _edge_mlp(h1m, h1p, dsqcol, wd, wp, m2w, m2b, p2w, p2b, be):
    e = h1m.shape[0]
    grid = e // be
    return pl.pallas_call(
        _edge_body,
        grid=(grid,),
        in_specs=[
            pl.BlockSpec((be, H), lambda i: (i, 0)),
            pl.BlockSpec((be, H), lambda i: (i, 0)),
            pl.BlockSpec((be, 1), lambda i: (i, 0)),
            pl.BlockSpec((1, H), lambda i: (0, 0)),
            pl.BlockSpec((1, H), lambda i: (0, 0)),
            pl.BlockSpec((H, H), lambda i: (0, 0)),
            pl.BlockSpec((1, H), lambda i: (0, 0)),
            pl.BlockSpec((H, H), lambda i: (0, 0)),
            pl.BlockSpec((1, H), lambda i: (0, 0)),
        ],
        out_specs=[
            pl.BlockSpec((be, H), lambda i: (i, 0)),
            pl.BlockSpec((be, H), lambda i: (i, 0)),
        ],
        out_shape=[
            jax.ShapeDtypeStruct((e, H), F32),
            jax.ShapeDtypeStruct((e, H), F32),
        ],
    )(h1m, h1p, dsqcol, wd, wp, m2w, m2b, p2w, p2b)


def _scatter_add(msg, msgp, rec, n):
    e = rec.shape[0]
    nblk = e // KE
    iters = (nblk + NS - 1) // NS
    zrows = 200
    nrblk = n // zrows
    riters = (nrblk + NS - 1) // NS
    mesh = plsc.VectorSubcoreMesh(core_axis_name="c", subcore_axis_name="s",
                                  num_cores=NC, num_subcores=NS)

    @functools.partial(
        pl.kernel,
        out_type=[
            jax.ShapeDtypeStruct((n, H), F32),
            jax.ShapeDtypeStruct((n, H), F32),
        ],
        mesh=mesh,
        scratch_types=[
            pltpu.VMEM((KE,), jnp.int32),
            pltpu.VMEM((KE, H), F32),
            pltpu.VMEM((zrows, H), F32),
            pltpu.VMEM_SHARED((n, H), F32),
        ],
    )
    def k(msg_hbm, msgp_hbm, rec_hbm, agg_hbm, aggp_hbm,
          ridx, mbuf, zbuf, acc):
        cid = lax.axis_index("c")
        sid = lax.axis_index("s")

        def zrow(j, _):
            for v in range(H // 16):
                zbuf[j, pl.ds(v * 16, 16)] = jnp.zeros((16,), F32)
            return 0

        lax.fori_loop(0, zrows, zrow, 0)

        def zblock(it, _):
            b = it * NS + sid

            @pl.when(b < nrblk)
            def _():
                pltpu.sync_copy(zbuf, acc.at[pl.ds(b * zrows, zrows)])
            return 0

        lax.fori_loop(0, riters, zblock, 0)
        plsc.subcore_barrier()

        def run(src_hbm):
            def block(it, _):
                b = it * NS + sid

                @pl.when(b < nblk)
                def _():
                    base = b * KE
                    pltpu.sync_copy(rec_hbm.at[pl.ds(base, KE)], ridx)
                    pltpu.sync_copy(src_hbm.at[pl.ds(base, KE)], mbuf)
                    pltpu.sync_copy(mbuf, acc.at[ridx], add=True)
                return 0

            lax.fori_loop(0, iters, block, 0)

        @pl.when(cid == 0)
        def _():
            run(msg_hbm)

        @pl.when(cid == 1)
        def _():
            run(msgp_hbm)

        plsc.subcore_barrier()

        def wblock(it, _):
            b = it * NS + sid

            @pl.when(b < nrblk)
            def _():
                sl = pl.ds(b * zrows, zrows)

                @pl.when(cid == 0)
                def _():
                    pltpu.sync_copy(acc.at[sl], agg_hbm.at[sl])

                @pl.when(cid == 1)
                def _():
                    pltpu.sync_copy(acc.at[sl], aggp_hbm.at[sl])
            return 0

        lax.fori_loop(0, riters, wblock, 0)

    return k(msg, msgp, rec)


def _update_body(x_ref, pe_ref, ag_ref, agp_ref, u1w_ref, u1b_ref, u2w_ref,
                 u2b_ref, q1w_ref, q1b_ref, q2w_ref, q2b_ref, up_ref, upp_ref):
    cat1 = jnp.concatenate([x_ref[...], pe_ref[...], ag_ref[...]], axis=1)
    z1 = jnp.dot(cat1, u1w_ref[...], preferred_element_type=F32) + u1b_ref[...]
    h1 = z1 * jax.nn.sigmoid(z1)
    up_ref[...] = jnp.dot(h1, u2w_ref[...],
                          preferred_element_type=F32) + u2b_ref[...]
    cat2 = jnp.concatenate([pe_ref[...], agp_ref[...]], axis=1)
    z2 = jnp.dot(cat2, q1w_ref[...], preferred_element_type=F32) + q1b_ref[...]
    h2 = jnp.tanh(z2)
    upp_ref[...] = jnp.tanh(
        jnp.dot(h2, q2w_ref[...], preferred_element_type=F32) + q2b_ref[...])


def _node_update(x, pe, aggr, aggr_pos, u1w, u1b, u2w, u2b, q1w, q1b, q2w,
                 q2b, bn):
    n = x.shape[0]
    grid = n // bn
    row = lambda i: (i, 0)
    full = lambda shape: pl.BlockSpec(shape, lambda i: (0, 0))
    return pl.pallas_call(
        _update_body,
        grid=(grid,),
        in_specs=[
            pl.BlockSpec((bn, H), row),
            pl.BlockSpec((bn, H), row),
            pl.BlockSpec((bn, H), row),
            pl.BlockSpec((bn, H), row),
            full((3 * H, H)),
            full((1, H)),
            full((H, H)),
            full((1, H)),
            full((2 * H, H)),
            full((1, H)),
            full((H, H)),
            full((1, H)),
        ],
        out_specs=[
            pl.BlockSpec((bn, H), row),
            pl.BlockSpec((bn, H), row),
        ],
        out_shape=[
            jax.ShapeDtypeStruct((n, H), F32),
            jax.ShapeDtypeStruct((n, H), F32),
        ],
    )(x, pe, aggr, aggr_pos, u1w, u1b, u2w, u2b, q1w, q1b, q2w, q2b)


def kernel(x, pos, pe, edge_index, m1_W, m1_b, m2_W, m2_b, p1_W, p1_b, p2_W,
           p2_b, u1_W, u1_b, u2_W, u2_b, q1_W, q1_b, q2_W, q2_b):
    n = x.shape[0]
    e = edge_index.shape[1]
    send = edge_index[0].astype(jnp.int32)
    rec = edge_index[1].astype(jnp.int32)

    ws = jnp.zeros((2 * H, 2 * H), F32)
    ws = ws.at[:H, :H].set(m1_W[0:H])
    ws = ws.at[H:, :H].set(m1_W[H:2 * H])
    ws = ws.at[H:, H:].set(p1_W[0:H])
    wr = jnp.zeros((2 * H, 2 * H), F32)
    wr = wr.at[:H, :H].set(m1_W[2 * H:3 * H])
    wr = wr.at[H:, :H].set(m1_W[3 * H:4 * H])
    wr = wr.at[H:, H:].set(p1_W[H:2 * H])
    br = jnp.concatenate([m1_b, p1_b]).reshape(1, 2 * H)
    px = pos[:, 0].astype(F32)
    py = pos[:, 1].astype(F32)
    pz = pos[:, 2].astype(F32)

    stab, rtab = _make_tables(x, pe, ws, wr, br, bn=1000)
    h1m, h1p, dsq = _gather_combine(stab, rtab, send.reshape(e // KG, KG),
                                    rec.reshape(e // KG, KG), px, py, pz)
    dsqcol = dsq.reshape(e, 1)
    wd = m1_W[4 * H].reshape(1, H)
    wp = p1_W[2 * H].reshape(1, H)
    msg, msgp = _edge_mlp(h1m, h1p, dsqcol, wd, wp, m2_W, m2_b.reshape(1, H), p2_W,
                          p2_b.reshape(1, H), be=2560)
    aggr, aggr_pos = _scatter_add(msg, msgp, rec, n)
    update, update_pe = _node_update(
        x, pe, aggr, aggr_pos, u1_W, u1_b.reshape(1, H), u2_W,
        u2_b.reshape(1, H), q1_W, q1_b.reshape(1, H), q2_W,
        q2_b.reshape(1, H), bn=1000)
    return (update, update_pe)


# speedup vs baseline: 1.4264x; 1.1503x over previous
import functools

import jax
import jax.numpy as jnp
from jax import lax
from jax.experimental import pallas as pl
from jax.experimental.pallas import tpu as pltpu
from jax.experimental.pallas import tpu_sc as plsc

F32 = jnp.float32
H = 128
TW = 256
NC = 2
NS = 16
NW = NC * NS
KE = 128


def _tables_body(x_ref, pe_ref, ws_ref, wr_ref, br_ref, s_ref, r_ref):
    inp = jnp.concatenate([x_ref[...], pe_ref[...]], axis=1)
    s_ref[...] = jnp.dot(inp, ws_ref[...], preferred_element_type=F32)
    r_ref[...] = jnp.dot(inp, wr_ref[...],
                         preferred_element_type=F32) + br_ref[...]


def _make_tables(x, pe, ws, wr, br, bn):
    n = x.shape[0]
    grid = n // bn
    return pl.pallas_call(
        _tables_body,
        grid=(grid,),
        in_specs=[
            pl.BlockSpec((bn, H), lambda i: (i, 0)),
            pl.BlockSpec((bn, H), lambda i: (i, 0)),
            pl.BlockSpec((2 * H, 2 * H), lambda i: (0, 0)),
            pl.BlockSpec((2 * H, 2 * H), lambda i: (0, 0)),
            pl.BlockSpec((1, 2 * H), lambda i: (0, 0)),
        ],
        out_specs=[
            pl.BlockSpec((bn, TW), lambda i: (i, 0)),
            pl.BlockSpec((bn, TW), lambda i: (i, 0)),
        ],
        out_shape=[
            jax.ShapeDtypeStruct((n, TW), F32),
            jax.ShapeDtypeStruct((n, TW), F32),
        ],
    )(x, pe, ws, wr, br)


KG = 80


def _gather_combine(stab, rtab, send2d, rec2d, px, py, pz):
    nblk = send2d.shape[0]
    e = nblk * KG
    n = px.shape[0]
    bpt = nblk // NW
    mesh = plsc.VectorSubcoreMesh(core_axis_name="c", subcore_axis_name="s",
                                  num_cores=NC, num_subcores=NS)

    @functools.partial(
        pl.kernel,
        out_type=[
            jax.ShapeDtypeStruct((e, H), F32),
            jax.ShapeDtypeStruct((e, H), F32),
            jax.ShapeDtypeStruct((e,), F32),
        ],
        mesh=mesh,
        compiler_params=pltpu.CompilerParams(needs_layout_passes=False),
        scratch_types=[
            [pltpu.VMEM((KG,), jnp.int32) for _ in range(2)],
            [pltpu.VMEM((KG,), jnp.int32) for _ in range(2)],
            [pltpu.VMEM((KG, TW), F32) for _ in range(2)],
            [pltpu.VMEM((KG, TW), F32) for _ in range(2)],
            [pltpu.VMEM((KG,), F32) for _ in range(2)],
            pltpu.VMEM((n,), F32),
            pltpu.VMEM((n,), F32),
            pltpu.VMEM((n,), F32),
            [pltpu.SemaphoreType.DMA for _ in range(2)],
            [pltpu.SemaphoreType.DMA for _ in range(2)],
            [pltpu.SemaphoreType.DMA for _ in range(2)],
        ],
    )
    def k(stab_hbm, rtab_hbm, send_hbm, rec_hbm, px_hbm, py_hbm, pz_hbm,
          outm_hbm, outp_hbm, dsq_hbm, sidx, ridx, bufa, bufb, dsqb,
          pxv, pyv, pzv, sem_i, sem_g, sem_w):
        wid = lax.axis_index("s") * NC + lax.axis_index("c")
        b0 = wid * bpt
        pltpu.sync_copy(px_hbm, pxv)
        pltpu.sync_copy(py_hbm, pyv)
        pltpu.sync_copy(pz_hbm, pzv)

        def issue_idx(b, t):
            pltpu.async_copy(send_hbm.at[b], sidx[t], sem_i[t])
            pltpu.async_copy(rec_hbm.at[b], ridx[t], sem_i[t])

        def wait_idx(b, t):
            pltpu.make_async_copy(send_hbm.at[b], sidx[t], sem_i[t]).wait()
            pltpu.make_async_copy(rec_hbm.at[b], ridx[t], sem_i[t]).wait()

        def issue_gather(t):
            pltpu.async_copy(stab_hbm.at[sidx[t]], bufa[t], sem_g[t])
            pltpu.async_copy(rtab_hbm.at[ridx[t]], bufb[t], sem_g[t])

        def wait_gather(t):
            pltpu.make_async_copy(stab_hbm.at[sidx[t]], bufa[t],
                                  sem_g[t]).wait()
            pltpu.make_async_copy(rtab_hbm.at[ridx[t]], bufb[t],
                                  sem_g[t]).wait()

        def issue_writes(b, t):
            base = b * KG
            pltpu.async_copy(bufa[t].at[:, pl.ds(0, H)],
                             outm_hbm.at[pl.ds(base, KG)], sem_w[t])
            pltpu.async_copy(bufa[t].at[:, pl.ds(H, H)],
                             outp_hbm.at[pl.ds(base, KG)], sem_w[t])
            pltpu.async_copy(dsqb[t], dsq_hbm.at[pl.ds(base, KG)], sem_w[t])

        def wait_writes(b, t):
            base = b * KG
            pltpu.make_async_copy(bufa[t].at[:, pl.ds(0, H)],
                                  outm_hbm.at[pl.ds(base, KG)],
                                  sem_w[t]).wait()
            pltpu.make_async_copy(bufa[t].at[:, pl.ds(H, H)],
                                  outp_hbm.at[pl.ds(base, KG)],
                                  sem_w[t]).wait()
            pltpu.make_async_copy(dsqb[t], dsq_hbm.at[pl.ds(base, KG)],
                                  sem_w[t]).wait()

        def compute(b, t):
            def dsq_chunk(j, _):
                sl = pl.ds(j * 16, 16)
                sv = sidx[t][sl]
                rv = ridx[t][sl]
                dx = plsc.load_gather(pxv, [sv]) - plsc.load_gather(pxv, [rv])
                dy = plsc.load_gather(pyv, [sv]) - plsc.load_gather(pyv, [rv])
                dz = plsc.load_gather(pzv, [sv]) - plsc.load_gather(pzv, [rv])
                dsqb[t][sl] = dx * dx + dy * dy + dz * dz
                return 0

            lax.fori_loop(0, KG // 16, dsq_chunk, 0)

            def add_row(j, _):
                for v in range(TW // 16):
                    sl = pl.ds(v * 16, 16)
                    bufa[t][j, sl] = bufa[t][j, sl] + bufb[t][j, sl]
                return 0

            lax.fori_loop(0, KG, add_row, 0)

        pltpu.sync_copy(send_hbm.at[b0], sidx[0])
        pltpu.sync_copy(rec_hbm.at[b0], ridx[0])
        issue_gather(0)
        issue_idx(b0 + 1, 1)

        def step(it, _):
            b = b0 + it

            def body(s, t):
                @pl.when(it + 1 < bpt)
                def _():
                    wait_idx(b + 1, t)

                @pl.when(it >= 1)
                def _():
                    wait_writes(b - 1, t)

                @pl.when(it + 1 < bpt)
                def _():
                    issue_gather(t)
                wait_gather(s)
                compute(b, s)
                issue_writes(b, s)

                @pl.when(it + 2 < bpt)
                def _():
                    issue_idx(b + 2, s)

            @pl.when(it % 2 == 0)
            def _():
                body(0, 1)

            @pl.when(it % 2 == 1)
            def _():
                body(1, 0)
            return 0

        lax.fori_loop(0, bpt, step, 0)
        wait_writes(b0 + bpt - 1, (bpt - 1) % 2)

    return k(stab, rtab, send2d, rec2d, px, py, pz)


def _edge_body(pm_ref, pp_ref, dsq_ref, wd_ref, wp_ref, m2w_ref, m2b_ref,
               p2w_ref, p2b_ref, msg_ref, msgp_ref):
    dist = jnp.sqrt(dsq_ref[...] + 1e-12)
    zm = pm_ref[...] + dist * wd_ref[...]
    zp = pp_ref[...] + dist * wp_ref[...]
    hm = zm * jax.nn.sigmoid(zm)
    t1 = jnp.dot(hm, m2w_ref[...], preferred_element_type=F32) + m2b_ref[...]
    msg_ref[...] = t1 * jax.nn.sigmoid(t1)
    hp = jnp.tanh(zp)
    t2 = jnp.dot(hp, p2w_ref[...], preferred_element_type=F32) + p2b_ref[...]
    msgp_ref[...] = jnp.tanh(t2)


def _edge_mlp(h1m, h1p, dsqcol, wd, wp, m2w, m2b, p2w, p2b, be):
    e = h1m.shape[0]
    grid = e // be
    return pl.pallas_call(
        _edge_body,
        grid=(grid,),
        in_specs=[
            pl.BlockSpec((be, H), lambda i: (i, 0)),
            pl.BlockSpec((be, H), lambda i: (i, 0)),
            pl.BlockSpec((be, 1), lambda i: (i, 0)),
            pl.BlockSpec((1, H), lambda i: (0, 0)),
            pl.BlockSpec((1, H), lambda i: (0, 0)),
            pl.BlockSpec((H, H), lambda i: (0, 0)),
            pl.BlockSpec((1, H), lambda i: (0, 0)),
            pl.BlockSpec((H, H), lambda i: (0, 0)),
            pl.BlockSpec((1, H), lambda i: (0, 0)),
        ],
        out_specs=[
            pl.BlockSpec((be, H), lambda i: (i, 0)),
            pl.BlockSpec((be, H), lambda i: (i, 0)),
        ],
        out_shape=[
            jax.ShapeDtypeStruct((e, H), F32),
            jax.ShapeDtypeStruct((e, H), F32),
        ],
    )(h1m, h1p, dsqcol, wd, wp, m2w, m2b, p2w, p2b)


def _scatter_add(msg, msgp, rec, n):
    e = rec.shape[0]
    nblk = e // KE
    iters = (nblk + NS - 1) // NS
    zrows = 8
    nzblk = n // zrows
    ziters = (nzblk + NS - 1) // NS
    wrows = 200
    nrblk = n // wrows
    riters = (nrblk + NS - 1) // NS
    mesh = plsc.VectorSubcoreMesh(core_axis_name="c", subcore_axis_name="s",
                                  num_cores=NC, num_subcores=NS)

    @functools.partial(
        pl.kernel,
        out_type=[
            jax.ShapeDtypeStruct((n, H), F32),
            jax.ShapeDtypeStruct((n, H), F32),
        ],
        mesh=mesh,
        scratch_types=[
            [pltpu.VMEM((KE,), jnp.int32) for _ in range(2)],
            [pltpu.VMEM((KE, H), F32) for _ in range(2)],
            pltpu.VMEM((zrows, H), F32),
            pltpu.VMEM_SHARED((n, H), F32),
            [pltpu.SemaphoreType.DMA for _ in range(2)],
            [pltpu.SemaphoreType.DMA for _ in range(2)],
        ],
    )
    def k(msg_hbm, msgp_hbm, rec_hbm, agg_hbm, aggp_hbm,
          ridx, mbuf, zbuf, acc, sem_p, sem_s):
        cid = lax.axis_index("c")
        sid = lax.axis_index("s")

        def zrow(j, _):
            for v in range(H // 16):
                zbuf[j, pl.ds(v * 16, 16)] = jnp.zeros((16,), F32)
            return 0

        lax.fori_loop(0, zrows, zrow, 0)

        def zblock(it, _):
            b = it * NS + sid

            @pl.when(b < nzblk)
            def _():
                pltpu.sync_copy(zbuf, acc.at[pl.ds(b * zrows, zrows)])
            return 0

        lax.fori_loop(0, ziters, zblock, 0)
        plsc.subcore_barrier()

        def run(src_hbm):
            def issue_pref(b, t):
                base = b * KE
                pltpu.async_copy(rec_hbm.at[pl.ds(base, KE)], ridx[t],
                                 sem_p[t])
                pltpu.async_copy(src_hbm.at[pl.ds(base, KE)], mbuf[t],
                                 sem_p[t])

            def wait_pref(b, t):
                base = b * KE
                pltpu.make_async_copy(rec_hbm.at[pl.ds(base, KE)], ridx[t],
                                      sem_p[t]).wait()
                pltpu.make_async_copy(src_hbm.at[pl.ds(base, KE)], mbuf[t],
                                      sem_p[t]).wait()

            def issue_scat(t):
                pltpu.async_copy(mbuf[t], acc.at[ridx[t]], sem_s[t], add=True)

            def wait_scat(t):
                pltpu.make_async_copy(mbuf[t], acc.at[ridx[t]],
                                      sem_s[t]).wait()

            @pl.when(sid < nblk)
            def _():
                issue_pref(sid, 0)

            def block(it, _):
                b = it * NS + sid

                def body(s, t):
                    @pl.when(b < nblk)
                    def _():
                        wait_pref(b, s)
                        issue_scat(s)

                    @pl.when((it >= 1) & (b - NS < nblk))
                    def _():
                        wait_scat(t)

                    @pl.when(b + NS < nblk)
                    def _():
                        issue_pref(b + NS, t)

                @pl.when(it % 2 == 0)
                def _():
                    body(0, 1)

                @pl.when(it % 2 == 1)
                def _():
                    body(1, 0)
                return 0

            lax.fori_loop(0, iters, block, 0)
            blast = (iters - 1) * NS + sid

            @pl.when(blast < nblk)
            def _():
                wait_scat((iters - 1) % 2)

        @pl.when(cid == 0)
        def _():
            run(msg_hbm)

        @pl.when(cid == 1)
        def _():
            run(msgp_hbm)

        plsc.subcore_barrier()

        def wblock(it, _):
            b = it * NS + sid

            @pl.when(b < nrblk)
            def _():
                sl = pl.ds(b * wrows, wrows)

                @pl.when(cid == 0)
                def _():
                    pltpu.sync_copy(acc.at[sl], agg_hbm.at[sl])

                @pl.when(cid == 1)
                def _():
                    pltpu.sync_copy(acc.at[sl], aggp_hbm.at[sl])
            return 0

        lax.fori_loop(0, riters, wblock, 0)

    return k(msg, msgp, rec)


def _update_body(x_ref, pe_ref, ag_ref, agp_ref, u1w_ref, u1b_ref, u2w_ref,
                 u2b_ref, q1w_ref, q1b_ref, q2w_ref, q2b_ref, up_ref, upp_ref):
    cat1 = jnp.concatenate([x_ref[...], pe_ref[...], ag_ref[...]], axis=1)
    z1 = jnp.dot(cat1, u1w_ref[...], preferred_element_type=F32) + u1b_ref[...]
    h1 = z1 * jax.nn.sigmoid(z1)
    up_ref[...] = jnp.dot(h1, u2w_ref[...],
                          preferred_element_type=F32) + u2b_ref[...]
    cat2 = jnp.concatenate([pe_ref[...], agp_ref[...]], axis=1)
    z2 = jnp.dot(cat2, q1w_ref[...], preferred_element_type=F32) + q1b_ref[...]
    h2 = jnp.tanh(z2)
    upp_ref[...] = jnp.tanh(
        jnp.dot(h2, q2w_ref[...], preferred_element_type=F32) + q2b_ref[...])


def _node_update(x, pe, aggr, aggr_pos, u1w, u1b, u2w, u2b, q1w, q1b, q2w,
                 q2b, bn):
    n = x.shape[0]
    grid = n // bn
    row = lambda i: (i, 0)
    full = lambda shape: pl.BlockSpec(shape, lambda i: (0, 0))
    return pl.pallas_call(
        _update_body,
        grid=(grid,),
        in_specs=[
            pl.BlockSpec((bn, H), row),
            pl.BlockSpec((bn, H), row),
            pl.BlockSpec((bn, H), row),
            pl.BlockSpec((bn, H), row),
            full((3 * H, H)),
            full((1, H)),
            full((H, H)),
            full((1, H)),
            full((2 * H, H)),
            full((1, H)),
            full((H, H)),
            full((1, H)),
        ],
        out_specs=[
            pl.BlockSpec((bn, H), row),
            pl.BlockSpec((bn, H), row),
        ],
        out_shape=[
            jax.ShapeDtypeStruct((n, H), F32),
            jax.ShapeDtypeStruct((n, H), F32),
        ],
    )(x, pe, aggr, aggr_pos, u1w, u1b, u2w, u2b, q1w, q1b, q2w, q2b)


def kernel(x, pos, pe, edge_index, m1_W, m1_b, m2_W, m2_b, p1_W, p1_b, p2_W,
           p2_b, u1_W, u1_b, u2_W, u2_b, q1_W, q1_b, q2_W, q2_b):
    n = x.shape[0]
    e = edge_index.shape[1]
    send = edge_index[0].astype(jnp.int32)
    rec = edge_index[1].astype(jnp.int32)

    ws = jnp.zeros((2 * H, 2 * H), F32)
    ws = ws.at[:H, :H].set(m1_W[0:H])
    ws = ws.at[H:, :H].set(m1_W[H:2 * H])
    ws = ws.at[H:, H:].set(p1_W[0:H])
    wr = jnp.zeros((2 * H, 2 * H), F32)
    wr = wr.at[:H, :H].set(m1_W[2 * H:3 * H])
    wr = wr.at[H:, :H].set(m1_W[3 * H:4 * H])
    wr = wr.at[H:, H:].set(p1_W[H:2 * H])
    br = jnp.concatenate([m1_b, p1_b]).reshape(1, 2 * H)
    px = pos[:, 0].astype(F32)
    py = pos[:, 1].astype(F32)
    pz = pos[:, 2].astype(F32)

    stab, rtab = _make_tables(x, pe, ws, wr, br, bn=1000)
    h1m, h1p, dsq = _gather_combine(stab, rtab, send.reshape(e // KG, KG),
                                    rec.reshape(e // KG, KG), px, py, pz)
    dsqcol = dsq.reshape(e, 1)
    wd = m1_W[4 * H].reshape(1, H)
    wp = p1_W[2 * H].reshape(1, H)
    msg, msgp = _edge_mlp(h1m, h1p, dsqcol, wd, wp, m2_W, m2_b.reshape(1, H), p2_W,
                          p2_b.reshape(1, H), be=2560)
    aggr, aggr_pos = _scatter_add(msg, msgp, rec, n)
    update, update_pe = _node_update(
        x, pe, aggr, aggr_pos, u1_W, u1_b.reshape(1, H), u2_W,
        u2_b.reshape(1, H), q1_W, q1_b.reshape(1, H), q2_W,
        q2_b.reshape(1, H), bn=1000)
    return (update, update_pe)


# speedup vs baseline: 1.5965x; 1.1192x over previous
import functools

import jax
import jax.numpy as jnp
from jax import lax
from jax.experimental import pallas as pl
from jax.experimental.pallas import tpu as pltpu
from jax.experimental.pallas import tpu_sc as plsc

F32 = jnp.float32
H = 128
TW = 256
NC = 2
NS = 16
NW = NC * NS
KE = 128


def _tables_body(x_ref, pe_ref, ws_ref, wr_ref, br_ref, s_ref, r_ref):
    inp = jnp.concatenate([x_ref[...], pe_ref[...]], axis=1)
    s_ref[...] = jnp.dot(inp, ws_ref[...], preferred_element_type=F32)
    r_ref[...] = jnp.dot(inp, wr_ref[...],
                         preferred_element_type=F32) + br_ref[...]


def _make_tables(x, pe, ws, wr, br, bn):
    n = x.shape[0]
    grid = n // bn
    return pl.pallas_call(
        _tables_body,
        grid=(grid,),
        in_specs=[
            pl.BlockSpec((bn, H), lambda i: (i, 0)),
            pl.BlockSpec((bn, H), lambda i: (i, 0)),
            pl.BlockSpec((2 * H, 2 * H), lambda i: (0, 0)),
            pl.BlockSpec((2 * H, 2 * H), lambda i: (0, 0)),
            pl.BlockSpec((1, 2 * H), lambda i: (0, 0)),
        ],
        out_specs=[
            pl.BlockSpec((bn, TW), lambda i: (i, 0)),
            pl.BlockSpec((bn, TW), lambda i: (i, 0)),
        ],
        out_shape=[
            jax.ShapeDtypeStruct((n, TW), F32),
            jax.ShapeDtypeStruct((n, TW), F32),
        ],
    )(x, pe, ws, wr, br)


KG = 80


def _gather_combine(stab, rtab, send2d, rec2d, px, py, pz):
    nblk = send2d.shape[0]
    e = nblk * KG
    n = px.shape[0]
    bpt = nblk // NW
    mesh = plsc.VectorSubcoreMesh(core_axis_name="c", subcore_axis_name="s",
                                  num_cores=NC, num_subcores=NS)

    @functools.partial(
        pl.kernel,
        out_type=[
            jax.ShapeDtypeStruct((e, H), F32),
            jax.ShapeDtypeStruct((e, H), F32),
            jax.ShapeDtypeStruct((e,), F32),
        ],
        mesh=mesh,
        compiler_params=pltpu.CompilerParams(needs_layout_passes=False),
        scratch_types=[
            [pltpu.VMEM((KG,), jnp.int32) for _ in range(2)],
            [pltpu.VMEM((KG,), jnp.int32) for _ in range(2)],
            [pltpu.VMEM((KG, TW), F32) for _ in range(2)],
            [pltpu.VMEM((KG, TW), F32) for _ in range(2)],
            [pltpu.VMEM((KG,), F32) for _ in range(2)],
            pltpu.VMEM((n,), F32),
            pltpu.VMEM((n,), F32),
            pltpu.VMEM((n,), F32),
            [pltpu.SemaphoreType.DMA for _ in range(2)],
            [pltpu.SemaphoreType.DMA for _ in range(2)],
            [pltpu.SemaphoreType.DMA for _ in range(2)],
        ],
    )
    def k(stab_hbm, rtab_hbm, send_hbm, rec_hbm, px_hbm, py_hbm, pz_hbm,
          outm_hbm, outp_hbm, dsq_hbm, sidx, ridx, bufa, bufb, dsqb,
          pxv, pyv, pzv, sem_i, sem_g, sem_w):
        wid = lax.axis_index("s") * NC + lax.axis_index("c")
        b0 = wid * bpt
        pltpu.sync_copy(px_hbm, pxv)
        pltpu.sync_copy(py_hbm, pyv)
        pltpu.sync_copy(pz_hbm, pzv)

        def issue_idx(b, t):
            pltpu.async_copy(send_hbm.at[b], sidx[t], sem_i[t])
            pltpu.async_copy(rec_hbm.at[b], ridx[t], sem_i[t])

        def wait_idx(b, t):
            pltpu.make_async_copy(send_hbm.at[b], sidx[t], sem_i[t]).wait()
            pltpu.make_async_copy(rec_hbm.at[b], ridx[t], sem_i[t]).wait()

        def issue_gather(t):
            pltpu.async_copy(stab_hbm.at[sidx[t]], bufa[t], sem_g[t])
            pltpu.async_copy(rtab_hbm.at[ridx[t]], bufb[t], sem_g[t])

        def wait_gather(t):
            pltpu.make_async_copy(stab_hbm.at[sidx[t]], bufa[t],
                                  sem_g[t]).wait()
            pltpu.make_async_copy(rtab_hbm.at[ridx[t]], bufb[t],
                                  sem_g[t]).wait()

        def issue_writes(b, t):
            base = b * KG
            pltpu.async_copy(bufa[t].at[:, pl.ds(0, H)],
                             outm_hbm.at[pl.ds(base, KG)], sem_w[t])
            pltpu.async_copy(bufa[t].at[:, pl.ds(H, H)],
                             outp_hbm.at[pl.ds(base, KG)], sem_w[t])
            pltpu.async_copy(dsqb[t], dsq_hbm.at[pl.ds(base, KG)], sem_w[t])

        def wait_writes(b, t):
            base = b * KG
            pltpu.make_async_copy(bufa[t].at[:, pl.ds(0, H)],
                                  outm_hbm.at[pl.ds(base, KG)],
                                  sem_w[t]).wait()
            pltpu.make_async_copy(bufa[t].at[:, pl.ds(H, H)],
                                  outp_hbm.at[pl.ds(base, KG)],
                                  sem_w[t]).wait()
            pltpu.make_async_copy(dsqb[t], dsq_hbm.at[pl.ds(base, KG)],
                                  sem_w[t]).wait()

        def compute(b, t):
            def dsq_chunk(j, _):
                sl = pl.ds(j * 16, 16)
                sv = sidx[t][sl]
                rv = ridx[t][sl]
                dx = plsc.load_gather(pxv, [sv]) - plsc.load_gather(pxv, [rv])
                dy = plsc.load_gather(pyv, [sv]) - plsc.load_gather(pyv, [rv])
                dz = plsc.load_gather(pzv, [sv]) - plsc.load_gather(pzv, [rv])
                dsqb[t][sl] = dx * dx + dy * dy + dz * dz
                return 0

            lax.fori_loop(0, KG // 16, dsq_chunk, 0)

            def add_row(j, _):
                for v in range(TW // 16):
                    sl = pl.ds(v * 16, 16)
                    bufa[t][j, sl] = bufa[t][j, sl] + bufb[t][j, sl]
                return 0

            lax.fori_loop(0, KG, add_row, 0)

        pltpu.sync_copy(send_hbm.at[b0], sidx[0])
        pltpu.sync_copy(rec_hbm.at[b0], ridx[0])
        issue_gather(0)
        issue_idx(b0 + 1, 1)

        def step(it, _):
            b = b0 + it

            def body(s, t):
                @pl.when(it + 1 < bpt)
                def _():
                    wait_idx(b + 1, t)

                @pl.when(it >= 1)
                def _():
                    wait_writes(b - 1, t)

                @pl.when(it + 1 < bpt)
                def _():
                    issue_gather(t)
                wait_gather(s)
                compute(b, s)
                issue_writes(b, s)

                @pl.when(it + 2 < bpt)
                def _():
                    issue_idx(b + 2, s)

            @pl.when(it % 2 == 0)
            def _():
                body(0, 1)

            @pl.when(it % 2 == 1)
            def _():
                body(1, 0)
            return 0

        lax.fori_loop(0, bpt, step, 0)
        wait_writes(b0 + bpt - 1, (bpt - 1) % 2)

    return k(stab, rtab, send2d, rec2d, px, py, pz)


def _edge_body(pm_ref, pp_ref, dsq_ref, wd_ref, wp_ref, m2w_ref, m2b_ref,
               p2w_ref, p2b_ref, msg_ref, msgp_ref):
    dist = jnp.sqrt(dsq_ref[...] + 1e-12)
    zm = pm_ref[...] + dist * wd_ref[...]
    zp = pp_ref[...] + dist * wp_ref[...]
    hm = zm * jax.nn.sigmoid(zm)
    t1 = jnp.dot(hm, m2w_ref[...], preferred_element_type=F32) + m2b_ref[...]
    msg_ref[...] = t1 * jax.nn.sigmoid(t1)
    hp = jnp.tanh(zp)
    t2 = jnp.dot(hp, p2w_ref[...], preferred_element_type=F32) + p2b_ref[...]
    msgp_ref[...] = jnp.tanh(t2)


def _edge_mlp(h1m, h1p, dsqcol, wd, wp, m2w, m2b, p2w, p2b, be):
    e = h1m.shape[0]
    grid = e // be
    return pl.pallas_call(
        _edge_body,
        grid=(grid,),
        in_specs=[
            pl.BlockSpec((be, H), lambda i: (i, 0)),
            pl.BlockSpec((be, H), lambda i: (i, 0)),
            pl.BlockSpec((be, 1), lambda i: (i, 0)),
            pl.BlockSpec((1, H), lambda i: (0, 0)),
            pl.BlockSpec((1, H), lambda i: (0, 0)),
            pl.BlockSpec((H, H), lambda i: (0, 0)),
            pl.BlockSpec((1, H), lambda i: (0, 0)),
            pl.BlockSpec((H, H), lambda i: (0, 0)),
            pl.BlockSpec((1, H), lambda i: (0, 0)),
        ],
        out_specs=[
            pl.BlockSpec((be, H), lambda i: (i, 0)),
            pl.BlockSpec((be, H), lambda i: (i, 0)),
        ],
        out_shape=[
            jax.ShapeDtypeStruct((e, H), F32),
            jax.ShapeDtypeStruct((e, H), F32),
        ],
    )(h1m, h1p, dsqcol, wd, wp, m2w, m2b, p2w, p2b)


def _scatter_add(msg, msgp, rec, n):
    e = rec.shape[0]
    nblk = e // KE
    iters = (nblk + NS - 1) // NS
    zrows = 8
    nzblk = n // zrows
    ziters = (nzblk + NS - 1) // NS
    wrows = 200
    nrblk = n // wrows
    riters = (nrblk + NS - 1) // NS
    mesh = plsc.VectorSubcoreMesh(core_axis_name="c", subcore_axis_name="s",
                                  num_cores=NC, num_subcores=NS)

    @functools.partial(
        pl.kernel,
        out_type=[
            jax.ShapeDtypeStruct((n, H), F32),
            jax.ShapeDtypeStruct((n, H), F32),
        ],
        mesh=mesh,
        scratch_types=[
            [pltpu.VMEM((KE,), jnp.int32) for _ in range(2)],
            [pltpu.VMEM((KE, H), F32) for _ in range(2)],
            pltpu.VMEM((zrows, H), F32),
            pltpu.VMEM_SHARED((n, H), F32),
            [pltpu.SemaphoreType.DMA for _ in range(2)],
            [pltpu.SemaphoreType.DMA for _ in range(2)],
        ],
    )
    def k(msg_hbm, msgp_hbm, rec_hbm, agg_hbm, aggp_hbm,
          ridx, mbuf, zbuf, acc, sem_p, sem_s):
        cid = lax.axis_index("c")
        sid = lax.axis_index("s")

        def zrow(j, _):
            for v in range(H // 16):
                zbuf[j, pl.ds(v * 16, 16)] = jnp.zeros((16,), F32)
            return 0

        lax.fori_loop(0, zrows, zrow, 0)

        def zblock(it, _):
            b = it * NS + sid

            @pl.when(b < nzblk)
            def _():
                pltpu.sync_copy(zbuf, acc.at[pl.ds(b * zrows, zrows)])
            return 0

        lax.fori_loop(0, ziters, zblock, 0)
        plsc.subcore_barrier()

        def run(src_hbm):
            def issue_pref(b, t):
                base = b * KE
                pltpu.async_copy(rec_hbm.at[pl.ds(base, KE)], ridx[t],
                                 sem_p[t])
                pltpu.async_copy(src_hbm.at[pl.ds(base, KE)], mbuf[t],
                                 sem_p[t])

            def wait_pref(b, t):
                base = b * KE
                pltpu.make_async_copy(rec_hbm.at[pl.ds(base, KE)], ridx[t],
                                      sem_p[t]).wait()
                pltpu.make_async_copy(src_hbm.at[pl.ds(base, KE)], mbuf[t],
                                      sem_p[t]).wait()

            def issue_scat(t):
                pltpu.async_copy(mbuf[t], acc.at[ridx[t]], sem_s[t], add=True)

            def wait_scat(t):
                pltpu.make_async_copy(mbuf[t], acc.at[ridx[t]],
                                      sem_s[t]).wait()

            @pl.when(sid < nblk)
            def _():
                issue_pref(sid, 0)

            def block(it, _):
                b = it * NS + sid

                def body(s, t):
                    @pl.when(b < nblk)
                    def _():
                        wait_pref(b, s)
                        issue_scat(s)

                    @pl.when((it >= 1) & (b - NS < nblk))
                    def _():
                        wait_scat(t)

                    @pl.when(b + NS < nblk)
                    def _():
                        issue_pref(b + NS, t)

                @pl.when(it % 2 == 0)
                def _():
                    body(0, 1)

                @pl.when(it % 2 == 1)
                def _():
                    body(1, 0)
                return 0

            lax.fori_loop(0, iters, block, 0)
            blast = (iters - 1) * NS + sid

            @pl.when(blast < nblk)
            def _():
                wait_scat((iters - 1) % 2)

        @pl.when(cid == 0)
        def _():
            run(msg_hbm)

        @pl.when(cid == 1)
        def _():
            run(msgp_hbm)

        plsc.subcore_barrier()

        def wblock(it, _):
            b = it * NS + sid

            @pl.when(b < nrblk)
            def _():
                sl = pl.ds(b * wrows, wrows)

                @pl.when(cid == 0)
                def _():
                    pltpu.sync_copy(acc.at[sl], agg_hbm.at[sl])

                @pl.when(cid == 1)
                def _():
                    pltpu.sync_copy(acc.at[sl], aggp_hbm.at[sl])
            return 0

        lax.fori_loop(0, riters, wblock, 0)

    return k(msg, msgp, rec)


def _update_body(x_ref, pe_ref, ag1_ref, ag2_ref, agp1_ref, agp2_ref,
                 u1w_ref, u1b_ref, u2w_ref, u2b_ref, q1w_ref, q1b_ref,
                 q2w_ref, q2b_ref, up_ref, upp_ref):
    ag = ag1_ref[...] + ag2_ref[...]
    agp = agp1_ref[...] + agp2_ref[...]
    cat1 = jnp.concatenate([x_ref[...], pe_ref[...], ag], axis=1)
    z1 = jnp.dot(cat1, u1w_ref[...], preferred_element_type=F32) + u1b_ref[...]
    h1 = z1 * jax.nn.sigmoid(z1)
    up_ref[...] = jnp.dot(h1, u2w_ref[...],
                          preferred_element_type=F32) + u2b_ref[...]
    cat2 = jnp.concatenate([pe_ref[...], agp], axis=1)
    z2 = jnp.dot(cat2, q1w_ref[...], preferred_element_type=F32) + q1b_ref[...]
    h2 = jnp.tanh(z2)
    upp_ref[...] = jnp.tanh(
        jnp.dot(h2, q2w_ref[...], preferred_element_type=F32) + q2b_ref[...])


def _node_update(x, pe, ag1, ag2, agp1, agp2, u1w, u1b, u2w, u2b, q1w, q1b,
                 q2w, q2b, bn):
    n = x.shape[0]
    grid = n // bn
    row = lambda i: (i, 0)
    full = lambda shape: pl.BlockSpec(shape, lambda i: (0, 0))
    return pl.pallas_call(
        _update_body,
        grid=(grid,),
        in_specs=[
            pl.BlockSpec((bn, H), row),
            pl.BlockSpec((bn, H), row),
            pl.BlockSpec((bn, H), row),
            pl.BlockSpec((bn, H), row),
            pl.BlockSpec((bn, H), row),
            pl.BlockSpec((bn, H), row),
            full((3 * H, H)),
            full((1, H)),
            full((H, H)),
            full((1, H)),
            full((2 * H, H)),
            full((1, H)),
            full((H, H)),
            full((1, H)),
        ],
        out_specs=[
            pl.BlockSpec((bn, H), row),
            pl.BlockSpec((bn, H), row),
        ],
        out_shape=[
            jax.ShapeDtypeStruct((n, H), F32),
            jax.ShapeDtypeStruct((n, H), F32),
        ],
    )(x, pe, ag1, ag2, agp1, agp2, u1w, u1b, u2w, u2b, q1w, q1b, q2w, q2b)


def kernel(x, pos, pe, edge_index, m1_W, m1_b, m2_W, m2_b, p1_W, p1_b, p2_W,
           p2_b, u1_W, u1_b, u2_W, u2_b, q1_W, q1_b, q2_W, q2_b):
    n = x.shape[0]
    e = edge_index.shape[1]
    send = edge_index[0].astype(jnp.int32)
    rec = edge_index[1].astype(jnp.int32)

    ws = jnp.zeros((2 * H, 2 * H), F32)
    ws = ws.at[:H, :H].set(m1_W[0:H])
    ws = ws.at[H:, :H].set(m1_W[H:2 * H])
    ws = ws.at[H:, H:].set(p1_W[0:H])
    wr = jnp.zeros((2 * H, 2 * H), F32)
    wr = wr.at[:H, :H].set(m1_W[2 * H:3 * H])
    wr = wr.at[H:, :H].set(m1_W[3 * H:4 * H])
    wr = wr.at[H:, H:].set(p1_W[H:2 * H])
    br = jnp.concatenate([m1_b, p1_b]).reshape(1, 2 * H)
    px = pos[:, 0].astype(F32)
    py = pos[:, 1].astype(F32)
    pz = pos[:, 2].astype(F32)

    stab, rtab = _make_tables(x, pe, ws, wr, br, bn=1000)
    wd = m1_W[4 * H].reshape(1, H)
    wp = p1_W[2 * H].reshape(1, H)

    e1 = (e // 2) // (NW * KG * 2) * (NW * KG * 2)
    bounds = [(0, e1), (e1, e)]
    aggs = []
    for lo, hi in bounds:
        eh = hi - lo
        sendh = lax.slice_in_dim(send, lo, hi)
        rech = lax.slice_in_dim(rec, lo, hi)
        h1m, h1p, dsq = _gather_combine(stab, rtab,
                                        sendh.reshape(eh // KG, KG),
                                        rech.reshape(eh // KG, KG),
                                        px, py, pz)
        msg, msgp = _edge_mlp(h1m, h1p, dsq.reshape(eh, 1), wd, wp, m2_W,
                              m2_b.reshape(1, H), p2_W, p2_b.reshape(1, H),
                              be=2560)
        aggs.append(_scatter_add(msg, msgp, rech, n))
    update, update_pe = _node_update(
        x, pe, aggs[0][0], aggs[1][0], aggs[0][1], aggs[1][1], u1_W,
        u1_b.reshape(1, H), u2_W, u2_b.reshape(1, H), q1_W,
        q1_b.reshape(1, H), q2_W, q2_b.reshape(1, H), bn=1000)
    return (update, update_pe)
